# Initial kernel scaffold; baseline (speedup 1.0000x reference)
import jax
import jax.numpy as jnp
from jax.experimental import pallas as pl


def kernel(text, table, W1, b1, W2, b2):
    raise NotImplementedError("write your pallas kernel here")


# speedup vs baseline: 1.0058x; 1.0058x over previous
import functools

import jax
import jax.numpy as jnp
from jax import lax
from jax.experimental import pallas as pl
from jax.experimental.pallas import tpu as pltpu
from jax.experimental.pallas import tpu_sc as plsc

NC = 2
NS = 16
NW = NC * NS

N_TOKENS = 1048576
DIM = 64
BATCH = 16
SEG = N_TOKENS // BATCH
TOK_PER_W = N_TOKENS // NW
CHUNK = 128
NCHUNK = TOK_PER_W // CHUNK
ROWS_PER_STEP = 4


def _sc_body(text_h, table_h, part_h, idx_v, rows_v, acc_v, sem0, sem1):
    wid = lax.axis_index("s") * NC + lax.axis_index("c")

    pltpu.sync_copy(text_h.at[wid], idx_v)

    pltpu.async_copy(table_h.at[idx_v.at[0]], rows_v.at[0], sem0)
    pltpu.async_copy(table_h.at[idx_v.at[1]], rows_v.at[1], sem1)

    def accum_chunk(b, acc):
        rv = rows_v.at[b]

        def inner(k, acc):
            a0, a1, a2, a3 = acc
            base = k * ROWS_PER_STEP
            for r in range(ROWS_PER_STEP):
                row = base + r
                a0 = a0 + rv[row, pl.ds(0, 16)]
                a1 = a1 + rv[row, pl.ds(16, 16)]
                a2 = a2 + rv[row, pl.ds(32, 16)]
                a3 = a3 + rv[row, pl.ds(48, 16)]
            return (a0, a1, a2, a3)

        return lax.fori_loop(0, CHUNK // ROWS_PER_STEP, inner, acc)

    def outer(i, acc):
        for b, sem in ((0, sem0), (1, sem1)):
            j = 2 * i + b
            pltpu.make_async_copy(
                table_h.at[idx_v.at[j]], rows_v.at[b], sem).wait()
            acc = accum_chunk(b, acc)

            @pl.when(j + 2 < NCHUNK)
            def _():
                pltpu.async_copy(
                    table_h.at[idx_v.at[j + 2]], rows_v.at[b], sem)
        return acc

    zeros = jnp.zeros((16,), jnp.float32)
    a0, a1, a2, a3 = lax.fori_loop(
        0, NCHUNK // 2, outer, (zeros, zeros, zeros, zeros))

    acc_v[pl.ds(0, 16)] = a0
    acc_v[pl.ds(16, 16)] = a1
    acc_v[pl.ds(32, 16)] = a2
    acc_v[pl.ds(48, 16)] = a3

    seg = wid // 2
    half = wid % 2
    pltpu.sync_copy(acc_v, part_h.at[half * BATCH + seg])


@functools.partial(jax.jit, static_argnames=())
def _sc_partial_sums(text3, table):
    mesh = plsc.VectorSubcoreMesh(
        core_axis_name="c", subcore_axis_name="s",
        num_cores=NC, num_subcores=NS)
    fn = pl.kernel(
        _sc_body,
        out_type=jax.ShapeDtypeStruct((NW, DIM), jnp.float32),
        mesh=mesh,
        scratch_types=[
            pltpu.VMEM((NCHUNK, CHUNK), jnp.int32),
            pltpu.VMEM((2, CHUNK, DIM), jnp.float32),
            pltpu.VMEM((DIM,), jnp.float32),
            pltpu.SemaphoreType.DMA,
            pltpu.SemaphoreType.DMA,
        ],
        compiler_params=pltpu.CompilerParams(use_tc_tiling_on_sc=False),
    )
    return fn(text3, table)


def _mlp_body(part_ref, w1t_ref, b1_ref, w2t_ref, b2_ref, out_ref):
    p = part_ref[...]
    pooled = (p[0] + p[1]) * (1.0 / SEG)
    h = lax.dot_general(pooled, w1t_ref[...], (((1,), (0,)), ((), ())),
                        preferred_element_type=jnp.float32)
    h = jnp.maximum(h + b1_ref[...], 0.0)
    o = lax.dot_general(h, w2t_ref[...], (((1,), (0,)), ((), ())),
                        preferred_element_type=jnp.float32)
    out_ref[...] = o + b2_ref[...]


def _mlp_call(part, w1t, b1, w2t, b2):
    return pl.pallas_call(
        _mlp_body,
        out_shape=jax.ShapeDtypeStruct((BATCH, w2t.shape[1]), jnp.float32),
    )(part, w1t, b1, w2t, b2)


def kernel(text, table, W1, b1, W2, b2):
    text3 = text.reshape(NW, NCHUNK, CHUNK)
    part = _sc_partial_sums(text3, table)
    part = part.reshape(2, BATCH, DIM)
    return _mlp_call(part, W1.T, b1.reshape(1, -1), W2.T, b2.reshape(1, -1))


# speedup vs baseline: 1.1158x; 1.1094x over previous
import functools

import jax
import jax.numpy as jnp
from jax import lax
from jax.experimental import pallas as pl
from jax.experimental.pallas import tpu as pltpu
from jax.experimental.pallas import tpu_sc as plsc

NC = 2
NS = 16
NW = NC * NS

N_TOKENS = 1048576
DIM = 64
BATCH = 16
SEG = N_TOKENS // BATCH
TOK_PER_W = N_TOKENS // NW
CHUNK = 128
NCHUNK = TOK_PER_W // CHUNK
NBUF = 8
ROWS_PER_STEP = 4


def _sc_body(text_h, table_h, part_h, idx_v, rows_v, acc_v, *sems):
    wid = lax.axis_index("s") * NC + lax.axis_index("c")

    pltpu.sync_copy(text_h.at[wid], idx_v)

    for b in range(NBUF):
        pltpu.async_copy(table_h.at[idx_v.at[b]], rows_v.at[b], sems[b])

    def accum_chunk(b, acc):
        rv = rows_v.at[b]

        def inner(k, acc):
            a0, a1, a2, a3 = acc
            base = k * ROWS_PER_STEP
            for r in range(ROWS_PER_STEP):
                row = base + r
                a0 = a0 + rv[row, pl.ds(0, 16)]
                a1 = a1 + rv[row, pl.ds(16, 16)]
                a2 = a2 + rv[row, pl.ds(32, 16)]
                a3 = a3 + rv[row, pl.ds(48, 16)]
            return (a0, a1, a2, a3)

        return lax.fori_loop(0, CHUNK // ROWS_PER_STEP, inner, acc)

    def outer(i, acc):
        for b in range(NBUF):
            j = i * NBUF + b
            pltpu.make_async_copy(
                table_h.at[idx_v.at[j]], rows_v.at[b], sems[b]).wait()
            acc = accum_chunk(b, acc)

            @pl.when(j + NBUF < NCHUNK)
            def _():
                pltpu.async_copy(
                    table_h.at[idx_v.at[j + NBUF]], rows_v.at[b], sems[b])
        return acc

    zeros = jnp.zeros((16,), jnp.float32)
    a0, a1, a2, a3 = lax.fori_loop(
        0, NCHUNK // NBUF, outer, (zeros, zeros, zeros, zeros))

    acc_v[pl.ds(0, 16)] = a0
    acc_v[pl.ds(16, 16)] = a1
    acc_v[pl.ds(32, 16)] = a2
    acc_v[pl.ds(48, 16)] = a3

    seg = wid // 2
    half = wid % 2
    pltpu.sync_copy(acc_v, part_h.at[half * BATCH + seg])


@functools.partial(jax.jit, static_argnames=())
def _sc_partial_sums(text3, table):
    mesh = plsc.VectorSubcoreMesh(
        core_axis_name="c", subcore_axis_name="s",
        num_cores=NC, num_subcores=NS)
    fn = pl.kernel(
        _sc_body,
        out_type=jax.ShapeDtypeStruct((NW, DIM), jnp.float32),
        mesh=mesh,
        scratch_types=[
            pltpu.VMEM((NCHUNK, CHUNK), jnp.int32),
            pltpu.VMEM((NBUF, CHUNK, DIM), jnp.float32),
            pltpu.VMEM((DIM,), jnp.float32),
        ] + [pltpu.SemaphoreType.DMA] * NBUF,
        compiler_params=pltpu.CompilerParams(use_tc_tiling_on_sc=False),
    )
    return fn(text3, table)


def _mlp_body(part_ref, w1t_ref, b1_ref, w2t_ref, b2_ref, out_ref):
    p = part_ref[...]
    pooled = (p[0] + p[1]) * (1.0 / SEG)
    h = lax.dot_general(pooled, w1t_ref[...], (((1,), (0,)), ((), ())),
                        preferred_element_type=jnp.float32)
    h = jnp.maximum(h + b1_ref[...], 0.0)
    o = lax.dot_general(h, w2t_ref[...], (((1,), (0,)), ((), ())),
                        preferred_element_type=jnp.float32)
    out_ref[...] = o + b2_ref[...]


def _mlp_call(part, w1t, b1, w2t, b2):
    return pl.pallas_call(
        _mlp_body,
        out_shape=jax.ShapeDtypeStruct((BATCH, w2t.shape[1]), jnp.float32),
    )(part, w1t, b1, w2t, b2)


def kernel(text, table, W1, b1, W2, b2):
    text3 = text.reshape(NW, NCHUNK, CHUNK)
    part = _sc_partial_sums(text3, table)
    part = part.reshape(2, BATCH, DIM)
    return _mlp_call(part, W1.T, b1.reshape(1, -1), W2.T, b2.reshape(1, -1))


# speedup vs baseline: 1.3956x; 1.2508x over previous
import functools

import jax
import jax.numpy as jnp
from jax import lax
from jax.experimental import pallas as pl
from jax.experimental.pallas import tpu as pltpu
from jax.experimental.pallas import tpu_sc as plsc

NC = 2
NS = 16
NW = NC * NS

N_TOKENS = 1048576
DIM = 64
BATCH = 16
SEG = N_TOKENS // BATCH
TOK_PER_W = N_TOKENS // NW
CHUNK = 128
NCHUNK = TOK_PER_W // CHUNK
NBUF = 8
ROWS_PER_STEP = 4

TCOLS = 2048
LASTB = (1000000 - 1) // (TCOLS // 2)


def _transpose_body(t0_ref, t1_ref, out_ref):
    out_ref[:, 0:DIM] = lax.transpose(t0_ref[...], (1, 0))
    out_ref[:, DIM:2 * DIM] = lax.transpose(t1_ref[...], (1, 0))


def _depad_table(tt):
    nrows = tt.shape[1]
    half = TCOLS // 2
    nblk = (nrows + TCOLS - 1) // TCOLS
    return pl.pallas_call(
        _transpose_body,
        grid=(nblk,),
        in_specs=[pl.BlockSpec((DIM, half),
                               lambda j: (0, jnp.minimum(2 * j, LASTB))),
                  pl.BlockSpec((DIM, half),
                               lambda j: (0, jnp.minimum(2 * j + 1, LASTB)))],
        out_specs=pl.BlockSpec((half, 2 * DIM), lambda j: (j, 0)),
        out_shape=jax.ShapeDtypeStruct((nblk * half, 2 * DIM), jnp.float32),
    )(tt, tt)


def _remap_indices(text):
    half = TCOLS // 2
    return (2 * (half * (text // TCOLS) + text % half)
            + (text // half) % 2)


def _sc_body(text_h, table_h, part_h, idx_v, rows_v, acc_v, *sems):
    wid = lax.axis_index("s") * NC + lax.axis_index("c")

    pltpu.sync_copy(text_h.at[wid], idx_v)

    for b in range(NBUF):
        pltpu.async_copy(table_h.at[idx_v.at[b]], rows_v.at[b], sems[b])

    def accum_chunk(b, acc):
        rv = rows_v.at[b]

        def inner(k, acc):
            a0, a1, a2, a3 = acc
            base = k * ROWS_PER_STEP
            for r in range(ROWS_PER_STEP):
                row = base + r
                a0 = a0 + rv[row, pl.ds(0, 16)]
                a1 = a1 + rv[row, pl.ds(16, 16)]
                a2 = a2 + rv[row, pl.ds(32, 16)]
                a3 = a3 + rv[row, pl.ds(48, 16)]
            return (a0, a1, a2, a3)

        return lax.fori_loop(0, CHUNK // ROWS_PER_STEP, inner, acc)

    def outer(i, acc):
        for b in range(NBUF):
            j = i * NBUF + b
            pltpu.make_async_copy(
                table_h.at[idx_v.at[j]], rows_v.at[b], sems[b]).wait()
            acc = accum_chunk(b, acc)

            @pl.when(j + NBUF < NCHUNK)
            def _():
                pltpu.async_copy(
                    table_h.at[idx_v.at[j + NBUF]], rows_v.at[b], sems[b])
        return acc

    zeros = jnp.zeros((16,), jnp.float32)
    a0, a1, a2, a3 = lax.fori_loop(
        0, NCHUNK // NBUF, outer, (zeros, zeros, zeros, zeros))

    acc_v[pl.ds(0, 16)] = a0
    acc_v[pl.ds(16, 16)] = a1
    acc_v[pl.ds(32, 16)] = a2
    acc_v[pl.ds(48, 16)] = a3

    seg = wid // 2
    half = wid % 2
    pltpu.sync_copy(acc_v, part_h.at[half * BATCH + seg])


@jax.jit
def _sc_partial_sums(text3, table):
    mesh = plsc.VectorSubcoreMesh(
        core_axis_name="c", subcore_axis_name="s",
        num_cores=NC, num_subcores=NS)
    fn = pl.kernel(
        _sc_body,
        out_type=jax.ShapeDtypeStruct((NW, DIM), jnp.float32),
        mesh=mesh,
        scratch_types=[
            pltpu.VMEM((NCHUNK, CHUNK), jnp.int32),
            pltpu.VMEM((NBUF, CHUNK, DIM), jnp.float32),
            pltpu.VMEM((DIM,), jnp.float32),
        ] + [pltpu.SemaphoreType.DMA] * NBUF,
        compiler_params=pltpu.CompilerParams(use_tc_tiling_on_sc=False),
    )
    return fn(text3, table)


def _mlp_body(part_ref, w1t_ref, b1_ref, w2t_ref, b2_ref, out_ref):
    p = part_ref[...]
    pooled = (p[0] + p[1]) * (1.0 / SEG)
    h = lax.dot_general(pooled, w1t_ref[...], (((1,), (0,)), ((), ())),
                        preferred_element_type=jnp.float32)
    h = jnp.maximum(h + b1_ref[...], 0.0)
    o = lax.dot_general(h, w2t_ref[...], (((1,), (0,)), ((), ())),
                        preferred_element_type=jnp.float32)
    out_ref[...] = o + b2_ref[...]


def _mlp_call(part, w1t, b1, w2t, b2):
    return pl.pallas_call(
        _mlp_body,
        out_shape=jax.ShapeDtypeStruct((BATCH, w2t.shape[1]), jnp.float32),
    )(part, w1t, b1, w2t, b2)


def kernel(text, table, W1, b1, W2, b2):
    table_lin = _depad_table(table.T)
    table_lin = table_lin.reshape(table_lin.shape[0] * 2, DIM)
    text3 = _remap_indices(text).reshape(NW, NCHUNK, CHUNK)
    part = _sc_partial_sums(text3, table_lin)
    part = part.reshape(2, BATCH, DIM)
    return _mlp_call(part, W1.T, b1.reshape(1, -1), W2.T, b2.reshape(1, -1))


# speedup vs baseline: 1.7848x; 1.2788x over previous
import functools

import jax
import jax.numpy as jnp
from jax import lax
from jax.experimental import pallas as pl
from jax.experimental.pallas import tpu as pltpu
from jax.experimental.pallas import tpu_sc as plsc

NC = 2
NS = 16
NW = NC * NS

N_TOKENS = 1048576
DIM = 64
BATCH = 16
SEG = N_TOKENS // BATCH
TOK_PER_W = N_TOKENS // NW
CHUNK = 128
NCHUNK = TOK_PER_W // CHUNK
NBUF = 8
ROWS_PER_STEP = 4

TCOLS = 4096
LASTB = (1000000 - 1) // (TCOLS // 2)


def _transpose_body(t0_ref, t1_ref, out_ref):
    out_ref[:, 0:DIM] = lax.transpose(t0_ref[...], (1, 0))
    out_ref[:, DIM:2 * DIM] = lax.transpose(t1_ref[...], (1, 0))


def _depad_table(tt):
    nrows = tt.shape[1]
    half = TCOLS // 2
    nblk = (nrows + TCOLS - 1) // TCOLS
    return pl.pallas_call(
        _transpose_body,
        grid=(nblk,),
        in_specs=[pl.BlockSpec((DIM, half),
                               lambda j: (0, jnp.minimum(2 * j, LASTB))),
                  pl.BlockSpec((DIM, half),
                               lambda j: (0, jnp.minimum(2 * j + 1, LASTB)))],
        out_specs=pl.BlockSpec((half, 2 * DIM), lambda j: (j, 0)),
        out_shape=jax.ShapeDtypeStruct((nblk * half, 2 * DIM), jnp.float32),
    )(tt, tt)


def _remap_indices(text):
    half = TCOLS // 2
    return (2 * (half * (text // TCOLS) + text % half)
            + (text // half) % 2)


def _sc_body(text_h, table_h, part_h, idx_v, rows_v, acc_v, *sems):
    wid = lax.axis_index("s") * NC + lax.axis_index("c")

    pltpu.sync_copy(text_h.at[wid], idx_v)

    for b in range(NBUF):
        pltpu.async_copy(table_h.at[idx_v.at[b]], rows_v.at[b], sems[b])

    def accum_chunk(b, acc):
        rv = rows_v.at[b]

        def inner(k, acc):
            a0, a1, a2, a3 = acc
            base = k * ROWS_PER_STEP
            for r in range(ROWS_PER_STEP):
                row = base + r
                a0 = a0 + rv[row, pl.ds(0, 16)]
                a1 = a1 + rv[row, pl.ds(16, 16)]
                a2 = a2 + rv[row, pl.ds(32, 16)]
                a3 = a3 + rv[row, pl.ds(48, 16)]
            return (a0, a1, a2, a3)

        return lax.fori_loop(0, CHUNK // ROWS_PER_STEP, inner, acc)

    def outer(i, acc):
        for b in range(NBUF):
            j = i * NBUF + b
            pltpu.make_async_copy(
                table_h.at[idx_v.at[j]], rows_v.at[b], sems[b]).wait()
            acc = accum_chunk(b, acc)

            @pl.when(j + NBUF < NCHUNK)
            def _():
                pltpu.async_copy(
                    table_h.at[idx_v.at[j + NBUF]], rows_v.at[b], sems[b])
        return acc

    zeros = jnp.zeros((16,), jnp.float32)
    a0, a1, a2, a3 = lax.fori_loop(
        0, NCHUNK // NBUF, outer, (zeros, zeros, zeros, zeros))

    acc_v[pl.ds(0, 16)] = a0
    acc_v[pl.ds(16, 16)] = a1
    acc_v[pl.ds(32, 16)] = a2
    acc_v[pl.ds(48, 16)] = a3

    seg = wid // 2
    half = wid % 2
    pltpu.sync_copy(acc_v, part_h.at[half * BATCH + seg])


@jax.jit
def _sc_partial_sums(text3, table):
    mesh = plsc.VectorSubcoreMesh(
        core_axis_name="c", subcore_axis_name="s",
        num_cores=NC, num_subcores=NS)
    fn = pl.kernel(
        _sc_body,
        out_type=jax.ShapeDtypeStruct((NW, DIM), jnp.float32),
        mesh=mesh,
        scratch_types=[
            pltpu.VMEM((NCHUNK, CHUNK), jnp.int32),
            pltpu.VMEM((NBUF, CHUNK, DIM), jnp.float32),
            pltpu.VMEM((DIM,), jnp.float32),
        ] + [pltpu.SemaphoreType.DMA] * NBUF,
        compiler_params=pltpu.CompilerParams(use_tc_tiling_on_sc=False),
    )
    return fn(text3, table)


def _mlp_body(part_ref, w1t_ref, b1_ref, w2t_ref, b2_ref, out_ref):
    p = part_ref[...]
    pooled = (p[0] + p[1]) * (1.0 / SEG)
    h = lax.dot_general(pooled, w1t_ref[...], (((1,), (0,)), ((), ())),
                        preferred_element_type=jnp.float32)
    h = jnp.maximum(h + b1_ref[...], 0.0)
    o = lax.dot_general(h, w2t_ref[...], (((1,), (0,)), ((), ())),
                        preferred_element_type=jnp.float32)
    out_ref[...] = o + b2_ref[...]


def _mlp_call(part, w1t, b1, w2t, b2):
    return pl.pallas_call(
        _mlp_body,
        out_shape=jax.ShapeDtypeStruct((BATCH, w2t.shape[1]), jnp.float32),
    )(part, w1t, b1, w2t, b2)


def kernel(text, table, W1, b1, W2, b2):
    table_lin = _depad_table(table.T)
    table_lin = table_lin.reshape(table_lin.shape[0] * 2, DIM)
    text3 = _remap_indices(text).reshape(NW, NCHUNK, CHUNK)
    part = _sc_partial_sums(text3, table_lin)
    part = part.reshape(2, BATCH, DIM)
    return _mlp_call(part, W1.T, b1.reshape(1, -1), W2.T, b2.reshape(1, -1))


# speedup vs baseline: 2.1050x; 1.1794x over previous
import functools

import jax
import jax.numpy as jnp
from jax import lax
from jax.experimental import pallas as pl
from jax.experimental.pallas import tpu as pltpu
from jax.experimental.pallas import tpu_sc as plsc

NC = 2
NS = 16
NW = NC * NS

N_TOKENS = 1048576
DIM = 64
BATCH = 16
SEG = N_TOKENS // BATCH
TOK_PER_W = N_TOKENS // NW
CHUNK = 128
NCHUNK = TOK_PER_W // CHUNK
NBUF = 8
ROWS_PER_STEP = 4

TCOLS = 8192
LASTB = (1000000 - 1) // (TCOLS // 2)


def _transpose_body(t0_ref, t1_ref, out_ref):
    out_ref[:, 0:DIM] = lax.transpose(t0_ref[...], (1, 0))
    out_ref[:, DIM:2 * DIM] = lax.transpose(t1_ref[...], (1, 0))


def _depad_table(tt):
    nrows = tt.shape[1]
    half = TCOLS // 2
    nblk = (nrows + TCOLS - 1) // TCOLS
    return pl.pallas_call(
        _transpose_body,
        grid=(nblk,),
        in_specs=[pl.BlockSpec((DIM, half),
                               lambda j: (0, jnp.minimum(2 * j, LASTB))),
                  pl.BlockSpec((DIM, half),
                               lambda j: (0, jnp.minimum(2 * j + 1, LASTB)))],
        out_specs=pl.BlockSpec((half, 2 * DIM), lambda j: (j, 0)),
        out_shape=jax.ShapeDtypeStruct((nblk * half, 2 * DIM), jnp.float32),
    )(tt, tt)


def _remap_indices(text):
    half = TCOLS // 2
    return (2 * (half * (text // TCOLS) + text % half)
            + (text // half) % 2)


def _sc_body(text_h, table_h, part_h, idx_v, rows_v, acc_v, *sems):
    wid = lax.axis_index("s") * NC + lax.axis_index("c")

    pltpu.sync_copy(text_h.at[wid], idx_v)

    for b in range(NBUF):
        pltpu.async_copy(table_h.at[idx_v.at[b]], rows_v.at[b], sems[b])

    def accum_chunk(b, acc):
        rv = rows_v.at[b]

        def inner(k, acc):
            a0, a1, a2, a3 = acc
            base = k * ROWS_PER_STEP
            for r in range(ROWS_PER_STEP):
                row = base + r
                a0 = a0 + rv[row, pl.ds(0, 16)]
                a1 = a1 + rv[row, pl.ds(16, 16)]
                a2 = a2 + rv[row, pl.ds(32, 16)]
                a3 = a3 + rv[row, pl.ds(48, 16)]
            return (a0, a1, a2, a3)

        return lax.fori_loop(0, CHUNK // ROWS_PER_STEP, inner, acc)

    def outer(i, acc):
        for b in range(NBUF):
            j = i * NBUF + b
            pltpu.make_async_copy(
                table_h.at[idx_v.at[j]], rows_v.at[b], sems[b]).wait()
            acc = accum_chunk(b, acc)

            @pl.when(j + NBUF < NCHUNK)
            def _():
                pltpu.async_copy(
                    table_h.at[idx_v.at[j + NBUF]], rows_v.at[b], sems[b])
        return acc

    zeros = jnp.zeros((16,), jnp.float32)
    a0, a1, a2, a3 = lax.fori_loop(
        0, NCHUNK // NBUF, outer, (zeros, zeros, zeros, zeros))

    acc_v[pl.ds(0, 16)] = a0
    acc_v[pl.ds(16, 16)] = a1
    acc_v[pl.ds(32, 16)] = a2
    acc_v[pl.ds(48, 16)] = a3

    seg = wid // 2
    half = wid % 2
    pltpu.sync_copy(acc_v, part_h.at[half * BATCH + seg])


@jax.jit
def _sc_partial_sums(text3, table):
    mesh = plsc.VectorSubcoreMesh(
        core_axis_name="c", subcore_axis_name="s",
        num_cores=NC, num_subcores=NS)
    fn = pl.kernel(
        _sc_body,
        out_type=jax.ShapeDtypeStruct((NW, DIM), jnp.float32),
        mesh=mesh,
        scratch_types=[
            pltpu.VMEM((NCHUNK, CHUNK), jnp.int32),
            pltpu.VMEM((NBUF, CHUNK, DIM), jnp.float32),
            pltpu.VMEM((DIM,), jnp.float32),
        ] + [pltpu.SemaphoreType.DMA] * NBUF,
        compiler_params=pltpu.CompilerParams(use_tc_tiling_on_sc=False),
    )
    return fn(text3, table)


def _mlp_body(part_ref, w1t_ref, b1_ref, w2t_ref, b2_ref, out_ref):
    p = part_ref[...]
    pooled = (p[0] + p[1]) * (1.0 / SEG)
    h = lax.dot_general(pooled, w1t_ref[...], (((1,), (0,)), ((), ())),
                        preferred_element_type=jnp.float32)
    h = jnp.maximum(h + b1_ref[...], 0.0)
    o = lax.dot_general(h, w2t_ref[...], (((1,), (0,)), ((), ())),
                        preferred_element_type=jnp.float32)
    out_ref[...] = o + b2_ref[...]


def _mlp_call(part, w1t, b1, w2t, b2):
    return pl.pallas_call(
        _mlp_body,
        out_shape=jax.ShapeDtypeStruct((BATCH, w2t.shape[1]), jnp.float32),
    )(part, w1t, b1, w2t, b2)


def kernel(text, table, W1, b1, W2, b2):
    table_lin = _depad_table(table.T)
    table_lin = table_lin.reshape(table_lin.shape[0] * 2, DIM)
    text3 = _remap_indices(text).reshape(NW, NCHUNK, CHUNK)
    part = _sc_partial_sums(text3, table_lin)
    part = part.reshape(2, BATCH, DIM)
    return _mlp_call(part, W1.T, b1.reshape(1, -1), W2.T, b2.reshape(1, -1))


# speedup vs baseline: 2.3054x; 1.0952x over previous
import functools

import jax
import jax.numpy as jnp
from jax import lax
from jax.experimental import pallas as pl
from jax.experimental.pallas import tpu as pltpu
from jax.experimental.pallas import tpu_sc as plsc

NC = 2
NS = 16
NW = NC * NS

N_TOKENS = 1048576
DIM = 64
BATCH = 16
SEG = N_TOKENS // BATCH
TOK_PER_W = N_TOKENS // NW
CHUNK = 128
NCHUNK = TOK_PER_W // CHUNK
NBUF = 8
ROWS_PER_STEP = 4

TCOLS = 16384
LASTB = (1000000 - 1) // (TCOLS // 2)


def _transpose_body(t0_ref, t1_ref, out_ref):
    out_ref[:, 0:DIM] = lax.transpose(t0_ref[...], (1, 0))
    out_ref[:, DIM:2 * DIM] = lax.transpose(t1_ref[...], (1, 0))


def _depad_table(tt):
    nrows = tt.shape[1]
    half = TCOLS // 2
    nblk = (nrows + TCOLS - 1) // TCOLS
    return pl.pallas_call(
        _transpose_body,
        grid=(nblk,),
        in_specs=[pl.BlockSpec((DIM, half),
                               lambda j: (0, jnp.minimum(2 * j, LASTB))),
                  pl.BlockSpec((DIM, half),
                               lambda j: (0, jnp.minimum(2 * j + 1, LASTB)))],
        out_specs=pl.BlockSpec((half, 2 * DIM), lambda j: (j, 0)),
        out_shape=jax.ShapeDtypeStruct((nblk * half, 2 * DIM), jnp.float32),
    )(tt, tt)


def _remap_indices(text):
    half = TCOLS // 2
    return (2 * (half * (text // TCOLS) + text % half)
            + (text // half) % 2)


def _sc_body(text_h, table_h, part_h, idx_v, rows_v, acc_v, *sems):
    wid = lax.axis_index("s") * NC + lax.axis_index("c")

    pltpu.sync_copy(text_h.at[wid], idx_v)

    for b in range(NBUF):
        pltpu.async_copy(table_h.at[idx_v.at[b]], rows_v.at[b], sems[b])

    def accum_chunk(b, acc):
        rv = rows_v.at[b]

        def inner(k, acc):
            a0, a1, a2, a3 = acc
            base = k * ROWS_PER_STEP
            for r in range(ROWS_PER_STEP):
                row = base + r
                a0 = a0 + rv[row, pl.ds(0, 16)]
                a1 = a1 + rv[row, pl.ds(16, 16)]
                a2 = a2 + rv[row, pl.ds(32, 16)]
                a3 = a3 + rv[row, pl.ds(48, 16)]
            return (a0, a1, a2, a3)

        return lax.fori_loop(0, CHUNK // ROWS_PER_STEP, inner, acc)

    def outer(i, acc):
        for b in range(NBUF):
            j = i * NBUF + b
            pltpu.make_async_copy(
                table_h.at[idx_v.at[j]], rows_v.at[b], sems[b]).wait()
            acc = accum_chunk(b, acc)

            @pl.when(j + NBUF < NCHUNK)
            def _():
                pltpu.async_copy(
                    table_h.at[idx_v.at[j + NBUF]], rows_v.at[b], sems[b])
        return acc

    zeros = jnp.zeros((16,), jnp.float32)
    a0, a1, a2, a3 = lax.fori_loop(
        0, NCHUNK // NBUF, outer, (zeros, zeros, zeros, zeros))

    acc_v[pl.ds(0, 16)] = a0
    acc_v[pl.ds(16, 16)] = a1
    acc_v[pl.ds(32, 16)] = a2
    acc_v[pl.ds(48, 16)] = a3

    seg = wid // 2
    half = wid % 2
    pltpu.sync_copy(acc_v, part_h.at[half * BATCH + seg])


@jax.jit
def _sc_partial_sums(text3, table):
    mesh = plsc.VectorSubcoreMesh(
        core_axis_name="c", subcore_axis_name="s",
        num_cores=NC, num_subcores=NS)
    fn = pl.kernel(
        _sc_body,
        out_type=jax.ShapeDtypeStruct((NW, DIM), jnp.float32),
        mesh=mesh,
        scratch_types=[
            pltpu.VMEM((NCHUNK, CHUNK), jnp.int32),
            pltpu.VMEM((NBUF, CHUNK, DIM), jnp.float32),
            pltpu.VMEM((DIM,), jnp.float32),
        ] + [pltpu.SemaphoreType.DMA] * NBUF,
        compiler_params=pltpu.CompilerParams(use_tc_tiling_on_sc=False),
    )
    return fn(text3, table)


def _mlp_body(part_ref, w1t_ref, b1_ref, w2t_ref, b2_ref, out_ref):
    p = part_ref[...]
    pooled = (p[0] + p[1]) * (1.0 / SEG)
    h = lax.dot_general(pooled, w1t_ref[...], (((1,), (0,)), ((), ())),
                        preferred_element_type=jnp.float32)
    h = jnp.maximum(h + b1_ref[...], 0.0)
    o = lax.dot_general(h, w2t_ref[...], (((1,), (0,)), ((), ())),
                        preferred_element_type=jnp.float32)
    out_ref[...] = o + b2_ref[...]


def _mlp_call(part, w1t, b1, w2t, b2):
    return pl.pallas_call(
        _mlp_body,
        out_shape=jax.ShapeDtypeStruct((BATCH, w2t.shape[1]), jnp.float32),
    )(part, w1t, b1, w2t, b2)


def kernel(text, table, W1, b1, W2, b2):
    table_lin = _depad_table(table.T)
    table_lin = table_lin.reshape(table_lin.shape[0] * 2, DIM)
    text3 = _remap_indices(text).reshape(NW, NCHUNK, CHUNK)
    part = _sc_partial_sums(text3, table_lin)
    part = part.reshape(2, BATCH, DIM)
    return _mlp_call(part, W1.T, b1.reshape(1, -1), W2.T, b2.reshape(1, -1))


# speedup vs baseline: 2.4140x; 1.0471x over previous
import functools

import jax
import jax.numpy as jnp
from jax import lax
from jax.experimental import pallas as pl
from jax.experimental.pallas import tpu as pltpu
from jax.experimental.pallas import tpu_sc as plsc

NC = 2
NS = 16
NW = NC * NS

N_TOKENS = 1048576
DIM = 64
BATCH = 16
SEG = N_TOKENS // BATCH
TOK_PER_W = N_TOKENS // NW
CHUNK = 128
NCHUNK = TOK_PER_W // CHUNK
NBUF = 8
ROWS_PER_STEP = 4

TCOLS = 32768
LASTB = (1000000 - 1) // (TCOLS // 2)


def _transpose_body(t0_ref, t1_ref, out_ref):
    out_ref[:, 0:DIM] = lax.transpose(t0_ref[...], (1, 0))
    out_ref[:, DIM:2 * DIM] = lax.transpose(t1_ref[...], (1, 0))


def _depad_table(tt):
    nrows = tt.shape[1]
    half = TCOLS // 2
    nblk = (nrows + TCOLS - 1) // TCOLS
    return pl.pallas_call(
        _transpose_body,
        grid=(nblk,),
        in_specs=[pl.BlockSpec((DIM, half),
                               lambda j: (0, jnp.minimum(2 * j, LASTB))),
                  pl.BlockSpec((DIM, half),
                               lambda j: (0, jnp.minimum(2 * j + 1, LASTB)))],
        out_specs=pl.BlockSpec((half, 2 * DIM), lambda j: (j, 0)),
        out_shape=jax.ShapeDtypeStruct((nblk * half, 2 * DIM), jnp.float32),
    )(tt, tt)


def _remap_indices(text):
    half = TCOLS // 2
    return (2 * (half * (text // TCOLS) + text % half)
            + (text // half) % 2)


def _sc_body(text_h, table_h, part_h, idx_v, rows_v, acc_v, *sems):
    wid = lax.axis_index("s") * NC + lax.axis_index("c")

    pltpu.sync_copy(text_h.at[wid], idx_v)

    for b in range(NBUF):
        pltpu.async_copy(table_h.at[idx_v.at[b]], rows_v.at[b], sems[b])

    def accum_chunk(b, acc):
        rv = rows_v.at[b]

        def inner(k, acc):
            a0, a1, a2, a3 = acc
            base = k * ROWS_PER_STEP
            for r in range(ROWS_PER_STEP):
                row = base + r
                a0 = a0 + rv[row, pl.ds(0, 16)]
                a1 = a1 + rv[row, pl.ds(16, 16)]
                a2 = a2 + rv[row, pl.ds(32, 16)]
                a3 = a3 + rv[row, pl.ds(48, 16)]
            return (a0, a1, a2, a3)

        return lax.fori_loop(0, CHUNK // ROWS_PER_STEP, inner, acc)

    def outer(i, acc):
        for b in range(NBUF):
            j = i * NBUF + b
            pltpu.make_async_copy(
                table_h.at[idx_v.at[j]], rows_v.at[b], sems[b]).wait()
            acc = accum_chunk(b, acc)

            @pl.when(j + NBUF < NCHUNK)
            def _():
                pltpu.async_copy(
                    table_h.at[idx_v.at[j + NBUF]], rows_v.at[b], sems[b])
        return acc

    zeros = jnp.zeros((16,), jnp.float32)
    a0, a1, a2, a3 = lax.fori_loop(
        0, NCHUNK // NBUF, outer, (zeros, zeros, zeros, zeros))

    acc_v[pl.ds(0, 16)] = a0
    acc_v[pl.ds(16, 16)] = a1
    acc_v[pl.ds(32, 16)] = a2
    acc_v[pl.ds(48, 16)] = a3

    seg = wid // 2
    half = wid % 2
    pltpu.sync_copy(acc_v, part_h.at[half * BATCH + seg])


@jax.jit
def _sc_partial_sums(text3, table):
    mesh = plsc.VectorSubcoreMesh(
        core_axis_name="c", subcore_axis_name="s",
        num_cores=NC, num_subcores=NS)
    fn = pl.kernel(
        _sc_body,
        out_type=jax.ShapeDtypeStruct((NW, DIM), jnp.float32),
        mesh=mesh,
        scratch_types=[
            pltpu.VMEM((NCHUNK, CHUNK), jnp.int32),
            pltpu.VMEM((NBUF, CHUNK, DIM), jnp.float32),
            pltpu.VMEM((DIM,), jnp.float32),
        ] + [pltpu.SemaphoreType.DMA] * NBUF,
        compiler_params=pltpu.CompilerParams(use_tc_tiling_on_sc=False),
    )
    return fn(text3, table)


def _mlp_body(part_ref, w1t_ref, b1_ref, w2t_ref, b2_ref, out_ref):
    p = part_ref[...]
    pooled = (p[0] + p[1]) * (1.0 / SEG)
    h = lax.dot_general(pooled, w1t_ref[...], (((1,), (0,)), ((), ())),
                        preferred_element_type=jnp.float32)
    h = jnp.maximum(h + b1_ref[...], 0.0)
    o = lax.dot_general(h, w2t_ref[...], (((1,), (0,)), ((), ())),
                        preferred_element_type=jnp.float32)
    out_ref[...] = o + b2_ref[...]


def _mlp_call(part, w1t, b1, w2t, b2):
    return pl.pallas_call(
        _mlp_body,
        out_shape=jax.ShapeDtypeStruct((BATCH, w2t.shape[1]), jnp.float32),
    )(part, w1t, b1, w2t, b2)


def kernel(text, table, W1, b1, W2, b2):
    table_lin = _depad_table(table.T)
    table_lin = table_lin.reshape(table_lin.shape[0] * 2, DIM)
    text3 = _remap_indices(text).reshape(NW, NCHUNK, CHUNK)
    part = _sc_partial_sums(text3, table_lin)
    part = part.reshape(2, BATCH, DIM)
    return _mlp_call(part, W1.T, b1.reshape(1, -1), W2.T, b2.reshape(1, -1))
